# baseline (speedup 1.0000x reference)
import functools

import jax
import jax.numpy as jnp
from jax import lax
from jax.experimental import pallas as pl
from jax.experimental.pallas import tpu as pltpu
from jax.experimental.pallas import tpu_sc as plsc

_L = 16


def _iota():
    return lax.iota(jnp.int32, _L)


def _splat(v):
    return jnp.full((_L,), v, jnp.int32)


def kernel(x, token_table, pos_table):
    B, T = x.shape
    V, D = token_table.shape
    assert pos_table.shape == (T, D)
    V2 = V // 2
    JNB = B // 128
    FULL_U = V // 128
    TAIL0 = FULL_U * 128
    NTAIL = V - TAIL0
    NW = 32

    xT = x.astype(jnp.int32).T
    ttT = token_table.T
    tail = jnp.pad(token_table[TAIL0:, :], ((0, 0), (0, 128 - D)))
    posp = jnp.pad(pos_table, ((0, 256 - T), (0, 128 - D)))

    mesh = plsc.VectorSubcoreMesh(core_axis_name="core", subcore_axis_name="subcore")
    cparams = pltpu.CompilerParams(needs_layout_passes=False)

    upairs = 123

    @functools.partial(
        pl.kernel,
        out_type=jax.ShapeDtypeStruct((V2, 128), jnp.float32),
        mesh=mesh,
        compiler_params=cparams,
        scratch_types=[
            pltpu.VMEM((64, 128), jnp.float32),
            pltpu.VMEM((64, 128), jnp.float32),
            pltpu.VMEM((64, 128), jnp.float32),
            pltpu.VMEM((64, 128), jnp.float32),
            pltpu.SemaphoreType.DMA,
            pltpu.SemaphoreType.DMA,
            pltpu.SemaphoreType.DMA,
            pltpu.SemaphoreType.DMA,
        ],
    )
    def format_table(ttT_hbm, tail_hbm, out_hbm, in0, in1, ob0, ob1,
                     s0, s1, so0, so1):
        wid = lax.axis_index("subcore") * 2 + lax.axis_index("core")
        ins = (in0, in1)
        obs = (ob0, ob1)
        sems = (s0, s1)
        osems = (so0, so1)

        def in_copy(j, p):
            return pltpu.make_async_copy(
                ttT_hbm.at[:, pl.ds(j * 128, 128)], ins[p], sems[p])

        def out_copy(j, p):
            return pltpu.make_async_copy(
                obs[p], out_hbm.at[pl.ds(j * 64, 64)], osems[p])

        def start_in(j, p):
            @pl.when(j < FULL_U)
            def _():
                in_copy(j, p).start()

        def do_unit(j, p):
            @pl.when(j < FULL_U)
            def _():
                in_copy(j, p).wait()

                @pl.when(j >= 2 * NW)
                def _():
                    out_copy(j - 2 * NW, p).wait()

                in_v = ins[p]
                out_v = obs[p]
                cols_c = [2 * _iota() + (2 * r0 + h)
                          for r0 in (0, 16, 32, 48) for h in (0, 1)]
                riota = [_iota() + r0 for r0 in (0, 16, 32, 48)]

                @plsc.parallel_loop(0, 16, unroll=4)
                def _k(k):
                    rotv = (_iota() + k) & 15
                    for icb, c0 in enumerate(range(0, 128, 16)):
                        rows = _splat(c0 % 64) | rotv
                        cv = _splat(c0) | rotv
                        h = c0 // 64
                        for irb, r0 in enumerate((0, 16, 32, 48)):
                            vals = plsc.load_gather(
                                in_v, [rows, cols_c[irb * 2 + h]])
                            plsc.store_scatter(out_v, [riota[irb], cv], vals)

                out_copy(j, p).start()
                start_in(j + 2 * NW, p)

        start_in(wid, 0)
        start_in(NW + wid, 1)

        @pl.loop(0, upairs)
        def _pair(i):
            for p in range(2):
                j = (2 * i + p) * NW + wid
                do_unit(j, p)

        jl0 = jnp.where(244 * NW + wid < FULL_U, 244 * NW + wid,
                        242 * NW + wid)
        out_copy(jl0, 0).wait()
        out_copy(243 * NW + wid, 1).wait()

        @pl.when(wid == 0)
        def _tail():
            pltpu.sync_copy(tail_hbm, in0)

            @pl.loop(0, NTAIL // 2)
            def _row(k):
                for cc in range(8):
                    src_r = 2 * k + (1 if cc >= 4 else 0)
                    sl = pl.ds((cc % 4) * 16, 16)
                    ob0[k, pl.ds(cc * 16, 16)] = in0[src_r, sl]

            pltpu.sync_copy(ob0.at[pl.ds(0, NTAIL // 2)],
                            out_hbm.at[pl.ds(V2 - NTAIL // 2, NTAIL // 2)])

    tok2 = format_table(ttT, tail)

    @functools.partial(
        pl.kernel,
        out_type=jax.ShapeDtypeStruct((T, 8, JNB, 8, 128), jnp.float32),
        mesh=mesh,
        compiler_params=cparams,
        scratch_types=[
            pltpu.VMEM((8, 128), jnp.int32),
            pltpu.VMEM((2, 128), jnp.int32),
            pltpu.VMEM((128, 128), jnp.float32),
            pltpu.VMEM((128, 128), jnp.float32),
            pltpu.VMEM((8, 8, 128), jnp.float32),
            pltpu.VMEM((8, 8, 128), jnp.float32),
            pltpu.VMEM((256, 128), jnp.float32),
            pltpu.SemaphoreType.DMA,
            pltpu.SemaphoreType.DMA,
            pltpu.SemaphoreType.DMA,
            pltpu.SemaphoreType.DMA,
            pltpu.SemaphoreType.DMA,
        ],
    )
    def embed(tok2_hbm, xT_hbm, pos_hbm, out_hbm, idx_v, k_v, rows0,
              rows1, oud0, oud1, pos_v, sp, g0, g1, so0, so1):
        wid = lax.axis_index("subcore") * 2 + lax.axis_index("core")
        pltpu.async_copy(pos_hbm, pos_v, sp).wait()
        rows = (rows0, rows1)
        ouds = (oud0, oud1)
        gsems = (g0, g1)
        osems = (so0, so1)

        def load_idx(t8):
            pltpu.sync_copy(
                xT_hbm.at[pl.ds(t8 * 8, 8), pl.ds(wid * 128, 128)], idx_v)

        def make_k(ti, p):
            @pl.loop(0, 8)
            def _mk(bc):
                vv = idx_v[ti, pl.ds(bc * 16, 16)]
                k_v[p, pl.ds(bc * 16, 16)] = lax.shift_right_logical(vv, 1)

        def gather_copy(p):
            return pltpu.make_async_copy(
                tok2_hbm.at[k_v.at[p]], rows[p], gsems[p])

        def out_copy(t, p):
            return pltpu.make_async_copy(
                ouds[p], out_hbm.at[t, :, wid], osems[p])

        def get_halves(ti):
            return [(idx_v[ti, pl.ds(bc * 16, 16)] & 1) * 64
                    for bc in range(8)]

        def compute(halves, t, p):
            rv = rows[p]
            gather_copy(p).wait()

            @pl.when(t >= 2)
            def _():
                out_copy(t - 2, p).wait()

            out_v = ouds[p]
            biota = [_iota() + bc * 16 for bc in range(8)]
            tspl = _splat(t)

            @plsc.parallel_loop(0, 16, unroll=4)
            def _k(k):
                rotv = (_iota() + k) & 15
                for c0 in range(0, 64, 16):
                    cadd = _splat(c0) | rotv
                    pv = plsc.load_gather(pos_v, [tspl, cadd])
                    c8 = lax.shift_right_logical(cadd, 3)
                    c1 = cadd & 7
                    for bc in range(8):
                        vals = plsc.load_gather(
                            rv, [biota[bc], halves[bc] | cadd])
                        plsc.store_scatter(out_v, [c8, c1, biota[bc]],
                                           vals + pv)

            out_copy(t, p).start()

        load_idx(0)
        make_k(0, 0)
        gather_copy(0).start()

        @pl.loop(0, T // 8)
        def _tblk(t8):
            for ti in range(8):
                t = t8 * 8 + ti
                p = ti & 1
                if ti < 7:
                    make_k(ti + 1, 1 - p)
                    gather_copy(1 - p).start()
                    compute(get_halves(ti), t, p)
                else:
                    halves7 = get_halves(7)

                    @pl.when(t8 + 1 < T // 8)
                    def _():
                        load_idx(t8 + 1)
                        make_k(0, 1 - p)
                        gather_copy(1 - p).start()

                    compute(halves7, t, p)

        out_copy(T - 2, 0).wait()
        out_copy(T - 1, 1).wait()

    out5 = embed(tok2, xT, posp)
    return out5.transpose(2, 4, 0, 1, 3).reshape(B, T, D)


# speedup vs baseline: 1.0737x; 1.0737x over previous
import functools

import jax
import jax.numpy as jnp
from jax import lax
from jax.experimental import pallas as pl
from jax.experimental.pallas import tpu as pltpu
from jax.experimental.pallas import tpu_sc as plsc

_L = 16


def _iota():
    return lax.iota(jnp.int32, _L)


def _splat(v):
    return jnp.full((_L,), v, jnp.int32)


def kernel(x, token_table, pos_table):
    B, T = x.shape
    V, D = token_table.shape
    assert pos_table.shape == (T, D)
    V2 = V // 2
    JNB = B // 128
    FULL_U = V // 128
    TAIL0 = FULL_U * 128
    NTAIL = V - TAIL0
    NW = 32

    xT = x.astype(jnp.int32).T
    ttT = token_table.T
    tail = jnp.pad(token_table[TAIL0:, :], ((0, 0), (0, 128 - D)))
    posp = jnp.pad(pos_table, ((0, 256 - T), (0, 128 - D)))

    mesh = plsc.VectorSubcoreMesh(core_axis_name="core", subcore_axis_name="subcore")
    cparams = pltpu.CompilerParams(needs_layout_passes=False)

    upairs = 123

    @functools.partial(
        pl.kernel,
        out_type=jax.ShapeDtypeStruct((V2, 128), jnp.float32),
        mesh=mesh,
        compiler_params=cparams,
        scratch_types=[
            pltpu.VMEM((64, 128), jnp.float32),
            pltpu.VMEM((64, 128), jnp.float32),
            pltpu.VMEM((64, 128), jnp.float32),
            pltpu.VMEM((64, 128), jnp.float32),
            pltpu.SemaphoreType.DMA,
            pltpu.SemaphoreType.DMA,
            pltpu.SemaphoreType.DMA,
            pltpu.SemaphoreType.DMA,
        ],
    )
    def format_table(ttT_hbm, tail_hbm, out_hbm, in0, in1, ob0, ob1,
                     s0, s1, so0, so1):
        wid = lax.axis_index("subcore") * 2 + lax.axis_index("core")
        ins = (in0, in1)
        obs = (ob0, ob1)
        sems = (s0, s1)
        osems = (so0, so1)

        def in_copy(j, p):
            return pltpu.make_async_copy(
                ttT_hbm.at[:, pl.ds(j * 128, 128)], ins[p], sems[p])

        def out_copy(j, p):
            return pltpu.make_async_copy(
                obs[p], out_hbm.at[pl.ds(j * 64, 64)], osems[p])

        def start_in(j, p):
            @pl.when(j < FULL_U)
            def _():
                in_copy(j, p).start()

        def do_unit(j, p):
            @pl.when(j < FULL_U)
            def _():
                in_copy(j, p).wait()

                @pl.when(j >= 2 * NW)
                def _():
                    out_copy(j - 2 * NW, p).wait()

                in_v = ins[p]
                out_v = obs[p]
                cols_c = [2 * _iota() + (2 * r0 + h)
                          for r0 in (0, 16, 32, 48) for h in (0, 1)]
                riota = [_iota() + r0 for r0 in (0, 16, 32, 48)]

                @plsc.parallel_loop(0, 16, unroll=2)
                def _k(k):
                    rotv = (_iota() + k) & 15
                    for icb, c0 in enumerate(range(0, 128, 16)):
                        rows = _splat(c0 % 64) | rotv
                        cv = _splat(c0) | rotv
                        h = c0 // 64
                        for irb, r0 in enumerate((0, 16, 32, 48)):
                            vals = plsc.load_gather(
                                in_v, [rows, cols_c[irb * 2 + h]])
                            plsc.store_scatter(out_v, [riota[irb], cv], vals)

                out_copy(j, p).start()
                start_in(j + 2 * NW, p)

        start_in(wid, 0)
        start_in(NW + wid, 1)

        @pl.loop(0, upairs)
        def _pair(i):
            for p in range(2):
                j = (2 * i + p) * NW + wid
                do_unit(j, p)

        jl0 = jnp.where(244 * NW + wid < FULL_U, 244 * NW + wid,
                        242 * NW + wid)
        out_copy(jl0, 0).wait()
        out_copy(243 * NW + wid, 1).wait()

        @pl.when(wid == 0)
        def _tail():
            pltpu.sync_copy(tail_hbm, in0)

            @pl.loop(0, NTAIL // 2)
            def _row(k):
                for cc in range(8):
                    src_r = 2 * k + (1 if cc >= 4 else 0)
                    sl = pl.ds((cc % 4) * 16, 16)
                    ob0[k, pl.ds(cc * 16, 16)] = in0[src_r, sl]

            pltpu.sync_copy(ob0.at[pl.ds(0, NTAIL // 2)],
                            out_hbm.at[pl.ds(V2 - NTAIL // 2, NTAIL // 2)])

    tok2 = format_table(ttT, tail)

    @functools.partial(
        pl.kernel,
        out_type=jax.ShapeDtypeStruct((T, 8, JNB, 8, 128), jnp.float32),
        mesh=mesh,
        compiler_params=cparams,
        scratch_types=[
            pltpu.VMEM((8, 128), jnp.int32),
            pltpu.VMEM((2, 128), jnp.int32),
            pltpu.VMEM((128, 128), jnp.float32),
            pltpu.VMEM((128, 128), jnp.float32),
            pltpu.VMEM((8, 8, 128), jnp.float32),
            pltpu.VMEM((8, 8, 128), jnp.float32),
            pltpu.VMEM((256, 128), jnp.float32),
            pltpu.SemaphoreType.DMA,
            pltpu.SemaphoreType.DMA,
            pltpu.SemaphoreType.DMA,
            pltpu.SemaphoreType.DMA,
            pltpu.SemaphoreType.DMA,
        ],
    )
    def embed(tok2_hbm, xT_hbm, pos_hbm, out_hbm, idx_v, k_v, rows0,
              rows1, oud0, oud1, pos_v, sp, g0, g1, so0, so1):
        wid = lax.axis_index("subcore") * 2 + lax.axis_index("core")
        pltpu.async_copy(pos_hbm, pos_v, sp).wait()
        rows = (rows0, rows1)
        ouds = (oud0, oud1)
        gsems = (g0, g1)
        osems = (so0, so1)

        def load_idx(t8):
            pltpu.sync_copy(
                xT_hbm.at[pl.ds(t8 * 8, 8), pl.ds(wid * 128, 128)], idx_v)

        def make_k(ti, p):
            @pl.loop(0, 8)
            def _mk(bc):
                vv = idx_v[ti, pl.ds(bc * 16, 16)]
                k_v[p, pl.ds(bc * 16, 16)] = lax.shift_right_logical(vv, 1)

        def gather_copy(p):
            return pltpu.make_async_copy(
                tok2_hbm.at[k_v.at[p]], rows[p], gsems[p])

        def out_copy(t, p):
            return pltpu.make_async_copy(
                ouds[p], out_hbm.at[t, :, wid], osems[p])

        def get_halves(ti):
            return [(idx_v[ti, pl.ds(bc * 16, 16)] & 1) * 64
                    for bc in range(8)]

        def compute(halves, t, p):
            rv = rows[p]
            gather_copy(p).wait()

            @pl.when(t >= 2)
            def _():
                out_copy(t - 2, p).wait()

            out_v = ouds[p]
            zero = _splat(0)
            sb = [(( _iota() + bc * 16) << 7) | halves[bc] for bc in range(8)]
            biota = [_iota() + bc * 16 for bc in range(8)]
            tspl = _splat(t)

            @plsc.parallel_loop(0, 16, unroll=2)
            def _k(k):
                rotv = (_iota() + k) & 15
                for c0 in range(0, 64, 16):
                    cadd = _splat(c0) | rotv
                    pv = plsc.load_gather(pos_v, [tspl, cadd])
                    dbase = cadd << 7
                    for bc in range(8):
                        vals = plsc.load_gather(rv, [zero, sb[bc] | cadd])
                        plsc.store_scatter(
                            out_v, [zero, zero, dbase | biota[bc]], vals + pv)

            out_copy(t, p).start()

        load_idx(0)
        make_k(0, 0)
        gather_copy(0).start()

        @pl.loop(0, T // 8)
        def _tblk(t8):
            for ti in range(8):
                t = t8 * 8 + ti
                p = ti & 1
                if ti < 7:
                    make_k(ti + 1, 1 - p)
                    gather_copy(1 - p).start()
                    compute(get_halves(ti), t, p)
                else:
                    halves7 = get_halves(7)

                    @pl.when(t8 + 1 < T // 8)
                    def _():
                        load_idx(t8 + 1)
                        make_k(0, 1 - p)
                        gather_copy(1 - p).start()

                    compute(halves7, t, p)

        out_copy(T - 2, 0).wait()
        out_copy(T - 1, 1).wait()

    out5 = embed(tok2, xT, posp)
    return out5.transpose(2, 4, 0, 1, 3).reshape(B, T, D)
